# Initial kernel scaffold; baseline (speedup 1.0000x reference)
import jax
import jax.numpy as jnp
from jax.experimental import pallas as pl


def kernel(output, target):
    raise NotImplementedError("write your pallas kernel here")


# speedup vs baseline: 2.9140x; 2.9140x over previous
import math

import jax
import jax.numpy as jnp
from jax.experimental import pallas as pl
from jax.experimental.pallas import tpu as pltpu

_NUM_CLASSES = 80
_NUM_ANCHORS = 3
_ANCHORS_W = (1.25, 2.0, 4.125)
_ANCHORS_H = (1.625, 3.75, 2.875)
_IGNORE_THRE = 0.5
_M = 50
_NCH = 5 + _NUM_CLASSES


def _clog(x):
    return jnp.maximum(jnp.log(x), -100.0)


def _loss_body(x_ref, p_ref, o_ref):
    a = pl.program_id(1)
    H = x_ref.shape[3]
    W = x_ref.shape[4]
    z = x_ref[0, 0]

    coli = jax.lax.broadcasted_iota(jnp.int32, (H, W), 1).astype(jnp.float32)
    rowi = jax.lax.broadcasted_iota(jnp.int32, (H, W), 0).astype(jnp.float32)

    af = a.astype(jnp.float32)
    aw = jnp.where(a == 0, _ANCHORS_W[0], jnp.where(a == 1, _ANCHORS_W[1], _ANCHORS_W[2]))
    ah = jnp.where(a == 0, _ANCHORS_H[0], jnp.where(a == 1, _ANCHORS_H[1], _ANCHORS_H[2]))

    sx = jax.nn.sigmoid(z[0])
    sy = jax.nn.sigmoid(z[1])
    zw = z[2]
    zh = z[3]
    conf = jax.nn.sigmoid(z[4])
    px = sx + coli
    py = sy + rowi
    pw = jnp.exp(zw) * aw
    ph = jnp.exp(zh) * ah
    parea = pw * ph

    zero = jnp.zeros((H, W), jnp.float32)

    def body_m(m, carry):
        miou, anyw, S, TXF, TYF, TWL, THL, CLSF = carry
        order = p_ref[0, 0, m]
        win = p_ref[0, 2, m]
        best = p_ref[0, 3, m]
        tif = p_ref[0, 4, m]
        tjf = p_ref[0, 5, m]
        cell = (rowi == tjf) & (coli == tif)
        upd = cell & ((win > 0.5) & (best == af))
        anyw = jnp.where(upd, 1.0, anyw)
        S = jnp.where(upd, p_ref[0, 10, m], S)
        TXF = jnp.where(upd, p_ref[0, 6, m], TXF)
        TYF = jnp.where(upd, p_ref[0, 7, m], TYF)
        TWL = jnp.where(upd, p_ref[0, 8, m], TWL)
        THL = jnp.where(upd, p_ref[0, 9, m], THL)
        CLSF = jnp.where(upd, p_ref[0, 11, m], CLSF)
        txm = p_ref[0, 12, m]
        tym = p_ref[0, 13, m]
        twm = p_ref[0, 14, m]
        thm = p_ref[0, 15, m]
        tlx = jnp.maximum(px - pw * 0.5, txm - twm * 0.5)
        tly = jnp.maximum(py - ph * 0.5, tym - thm * 0.5)
        brx = jnp.minimum(px + pw * 0.5, txm + twm * 0.5)
        bry = jnp.minimum(py + ph * 0.5, tym + thm * 0.5)
        en = (tlx < brx) & (tly < bry)
        ai = jnp.where(en, (brx - tlx) * (bry - tly), 0.0)
        iou = ai / (parea + twm * thm - ai)
        miou = jnp.where(order > 0.5, jnp.maximum(miou, iou), miou)
        return miou, anyw, S, TXF, TYF, TWL, THL, CLSF

    init = (jnp.full((H, W), -1e30, jnp.float32), zero, zero, zero, zero, zero, zero, zero)
    miou, anywf, S, TXF, TYF, TWL, THL, CLSF = jax.lax.fori_loop(0, _M, body_m, init)
    anyw = anywf > 0.5

    active = p_ref[0, 16, 0]
    pbest = miou > _IGNORE_THRE
    m_dense = jnp.where(active > 0.5, jnp.where(pbest, 0.0, 1.0), 1.0)
    l_obj = jnp.where(anyw, -_clog(conf),
                      jnp.where(m_dense > 0.5, -_clog(1.0 - conf), 0.0))
    s2 = S * S
    l_xy = jnp.where(
        anyw,
        -(TXF * _clog(sx) + (1.0 - TXF) * _clog(1.0 - sx)) * s2
        - (TYF * _clog(sy) + (1.0 - TYF) * _clog(1.0 - sy)) * s2,
        0.0)
    l_wh = jnp.where(anyw, 0.5 * s2 * ((zw - TWL) ** 2 + (zh - THL) ** 2), 0.0)

    zc = z[5:]
    pc = jax.nn.sigmoid(zc)
    ciota = jax.lax.broadcasted_iota(jnp.int32, (_NUM_CLASSES, H, W), 0).astype(jnp.float32)
    tc = jnp.where((ciota == CLSF[None]) & anyw[None], 1.0, 0.0)
    l_cls = jnp.where(anyw[None],
                      -(tc * _clog(pc) + (1.0 - tc) * _clog(1.0 - pc)), 0.0)

    total = jnp.sum(l_obj) + jnp.sum(l_xy) + jnp.sum(l_wh) + jnp.sum(l_cls)
    o_ref[0, 0] = jnp.full((8, 128), total, jnp.float32)


def _prep(labels, nW, nH):
    B = labels.shape[0]
    aw = jnp.asarray(_ANCHORS_W, jnp.float32)
    ah = jnp.asarray(_ANCHORS_H, jnp.float32)
    nlabel = jnp.sum(jnp.sum(labels, axis=2) > 0, axis=1)
    order = jnp.arange(_M)[None, :] < nlabel[:, None]
    tx = labels[:, :, 1] * nW
    ty = labels[:, :, 2] * nH
    tw = labels[:, :, 3] * nW
    th = labels[:, :, 4] * nH
    ti = tx.astype(jnp.int16).astype(jnp.int32)
    tj = ty.astype(jnp.int16).astype(jnp.int32)
    twx = tw[:, :, None]
    thx = th[:, :, None]
    awx = aw[None, None, :]
    ahx = ah[None, None, :]
    brx = jnp.minimum(twx, awx)
    bry = jnp.minimum(thx, ahx)
    en = ((brx > 0.0) & (bry > 0.0)).astype(jnp.float32)
    area_i = brx * bry * en
    area_u = twx * thx + awx * ahx - area_i
    iou = area_i / area_u
    rho2 = ((twx - awx) ** 2 + (thx - ahx) ** 2) / 4.0
    c2 = jnp.maximum(twx, awx) ** 2 + jnp.maximum(thx, ahx) ** 2 + 1e-16
    v = 4.0 / (math.pi ** 2) * (jnp.arctan(twx / thx) - jnp.arctan(awx / ahx)) ** 2
    alpha = v / (1.0 - iou + v)
    ciou = iou - (rho2 / c2 + v * alpha)
    best_all = jnp.argmax(ciou, axis=-1)
    best = best_all % _NUM_ANCHORS
    best_mask = best_all < _NUM_ANCHORS
    active = (nlabel > 0) & jnp.any(best_mask & order, axis=1)
    write = order & best_mask & active[:, None]
    same = ((best[:, :, None] == best[:, None, :])
            & (ti[:, :, None] == ti[:, None, :])
            & (tj[:, :, None] == tj[:, None, :]))
    mlt = jnp.arange(_M)[:, None] < jnp.arange(_M)[None, :]
    clobbered = jnp.any(write[:, None, :] & same & mlt[None], axis=2)
    winner = write & ~clobbered
    txf = tx - ti.astype(jnp.float32)
    tyf = ty - tj.astype(jnp.float32)
    twl = jnp.log(tw / aw[best] + 1e-16)
    thl = jnp.log(th / ah[best] + 1e-16)
    scale = jnp.sqrt(2.0 - tw * th / (nW * nH))
    cls = labels[:, :, 0].astype(jnp.int16).astype(jnp.int32)
    f32 = jnp.float32
    packed = jnp.stack([
        order.astype(f32), write.astype(f32), winner.astype(f32),
        best.astype(f32), ti.astype(f32), tj.astype(f32),
        txf, tyf, twl, thl, scale, cls.astype(f32),
        tx, ty, tw, th,
        jnp.broadcast_to(active.astype(f32)[:, None], (B, _M)),
    ], axis=1)
    return packed


def kernel(output, target):
    B, C, H, W = output.shape
    packed = _prep(target, W, H)
    out5 = output.reshape(B, _NUM_ANCHORS, _NCH, H, W)
    partial = pl.pallas_call(
        _loss_body,
        grid=(B, _NUM_ANCHORS),
        in_specs=[
            pl.BlockSpec((1, 1, _NCH, H, W), lambda b, a: (b, a, 0, 0, 0)),
            pl.BlockSpec((1, 17, _M), lambda b, a: (b, 0, 0),
                         memory_space=pltpu.SMEM),
        ],
        out_specs=pl.BlockSpec((1, 1, 8, 128), lambda b, a: (b, a, 0, 0)),
        out_shape=jax.ShapeDtypeStruct((B, _NUM_ANCHORS, 8, 128), jnp.float32),
    )(out5, packed)
    return jnp.sum(partial[:, :, 0, 0]) / B
